# Initial kernel scaffold; baseline (speedup 1.0000x reference)
import functools

import jax
import jax.numpy as jnp
from jax import lax
from jax.experimental import pallas as pl
from jax.experimental.pallas import tpu as pltpu
from jax.experimental.pallas import tpu_sc as plsc

_N_CLS = 1024
_N_CTX = 16
_DIM = 768
_SEQ = 77
_NC = 2
_NS = 16
_NW = _NC * _NS
_CPW = _N_CLS // _NW
_NIDX = 64
_NSUF = _SEQ - 1 - _N_CTX


_mesh = plsc.VectorSubcoreMesh(core_axis_name="c", subcore_axis_name="s")


@functools.partial(
    pl.kernel,
    mesh=_mesh,
    out_type=jax.ShapeDtypeStruct((_N_CLS, _SEQ, _DIM), jnp.float32),
    scratch_types=[
        pltpu.VMEM((_CPW, _SEQ), jnp.int32),
        pltpu.VMEM((_NIDX,), jnp.int32),
        pltpu.VMEM((_NIDX, _DIM), jnp.float32),
        pltpu.VMEM((_N_CTX, _DIM), jnp.float32),
        pltpu.SemaphoreType.DMA,
    ],
)
def _prompt_kernel(tok_hbm, table_hbm, ctx_hbm, out_hbm,
                   tok_v, idx_v, rows_v, ctx_v, sem):
    wid = lax.axis_index("s") * _NC + lax.axis_index("c")
    base_c = wid * _CPW
    pltpu.sync_copy(tok_hbm.at[pl.ds(base_c, _CPW)], tok_v)
    pltpu.sync_copy(ctx_hbm, ctx_v)

    i16 = lax.iota(jnp.int32, 16)
    pos = [
        jnp.where((i16 + 16 * j) == 0, 0,
                  jnp.minimum(i16 + 16 * j + _N_CTX, _SEQ - 1))
        for j in range(_NIDX // 16)
    ]

    def body(ci, carry):
        cvec = jnp.full((16,), ci, jnp.int32)
        for j in range(_NIDX // 16):
            idx_v[pl.ds(16 * j, 16)] = plsc.load_gather(tok_v, [cvec, pos[j]])
        pltpu.async_copy(table_hbm.at[idx_v], rows_v, sem).wait()
        c = base_c + ci
        pltpu.sync_copy(rows_v.at[pl.ds(0, 1)], out_hbm.at[c, pl.ds(0, 1)])
        pltpu.sync_copy(ctx_v, out_hbm.at[c, pl.ds(1, _N_CTX)])
        pltpu.sync_copy(rows_v.at[pl.ds(1, _NSUF)],
                        out_hbm.at[c, pl.ds(1 + _N_CTX, _NSUF)])
        return carry

    lax.fori_loop(0, _CPW, body, 0)


def kernel(tokenized_prompts, token_embedding, ctx):
    return _prompt_kernel(tokenized_prompts, token_embedding,
                          ctx.astype(jnp.float32))


# speedup vs baseline: 1.4441x; 1.4441x over previous
import functools

import jax
import jax.numpy as jnp
from jax import lax
from jax.experimental import pallas as pl
from jax.experimental.pallas import tpu as pltpu
from jax.experimental.pallas import tpu_sc as plsc

_N_CLS = 1024
_N_CTX = 16
_DIM = 768
_SEQ = 77
_NC = 2
_NS = 16
_NW = _NC * _NS
_CPW = _N_CLS // _NW
_NSUF = _SEQ - 1 - _N_CTX
_NIDX = 1 + _NSUF


_mesh = plsc.VectorSubcoreMesh(core_axis_name="c", subcore_axis_name="s")


@functools.partial(
    pl.kernel,
    mesh=_mesh,
    out_type=jax.ShapeDtypeStruct((_N_CLS, _SEQ, _DIM), jnp.float32),
    scratch_types=[
        pltpu.VMEM((_CPW, _SEQ), jnp.int32),
        pltpu.VMEM((64,), jnp.int32),
        pltpu.VMEM((_SEQ, _DIM), jnp.float32),
        pltpu.VMEM((1, _DIM), jnp.float32),
        pltpu.SemaphoreType.DMA,
    ],
    compiler_params=pltpu.CompilerParams(use_tc_tiling_on_sc=False,
                                         needs_layout_passes=False),
)
def _prompt_kernel(tok_hbm, table_hbm, ctx_hbm, out_hbm,
                   tok_v, idx_v, buf_v, ctx15_v, sem):
    wid = lax.axis_index("s") * _NC + lax.axis_index("c")
    base_c = wid * _CPW
    pltpu.sync_copy(tok_hbm.at[pl.ds(base_c, _CPW)], tok_v)
    pltpu.sync_copy(ctx_hbm, buf_v.at[pl.ds(1, _N_CTX)])
    pltpu.sync_copy(ctx_hbm.at[pl.ds(_N_CTX - 1, 1)], ctx15_v)

    i16 = lax.iota(jnp.int32, 16)
    pos = [
        jnp.where((i16 + 16 * j) == 0, 0,
                  jnp.minimum(i16 + 16 * j + _N_CTX, _SEQ - 1))
        for j in range(4)
    ]

    def body(ci, carry):
        cvec = jnp.full((16,), ci, jnp.int32)
        for j in range(4):
            idx_v[pl.ds(16 * j, 16)] = plsc.load_gather(tok_v, [cvec, pos[j]])
        pltpu.async_copy(table_hbm.at[idx_v.at[pl.ds(0, _NIDX)]],
                         buf_v.at[pl.ds(_N_CTX, _NIDX)], sem).wait()
        for k in range(_DIM // 16):
            sl = pl.ds(16 * k, 16)
            buf_v[0, sl] = buf_v[_N_CTX, sl]
            buf_v[_N_CTX, sl] = ctx15_v[0, sl]
        pltpu.sync_copy(buf_v, out_hbm.at[base_c + ci])
        return carry

    lax.fori_loop(0, _CPW, body, 0)


def kernel(tokenized_prompts, token_embedding, ctx):
    return _prompt_kernel(tokenized_prompts, token_embedding,
                          ctx.astype(jnp.float32))


# speedup vs baseline: 1.4717x; 1.0191x over previous
import functools

import jax
import jax.numpy as jnp
from jax import lax
from jax.experimental import pallas as pl
from jax.experimental.pallas import tpu as pltpu
from jax.experimental.pallas import tpu_sc as plsc

_N_CLS = 1024
_N_CTX = 16
_DIM = 768
_SEQ = 77
_NC = 2
_NS = 16
_NW = _NC * _NS
_CPW = _N_CLS // _NW
_NSUF = _SEQ - 1 - _N_CTX
_NIDX = 1 + _NSUF


_mesh = plsc.VectorSubcoreMesh(core_axis_name="c", subcore_axis_name="s")


@functools.partial(
    pl.kernel,
    mesh=_mesh,
    out_type=jax.ShapeDtypeStruct((_N_CLS, _SEQ, _DIM), jnp.float32),
    scratch_types=[
        pltpu.VMEM((_CPW, _SEQ), jnp.int32),
        pltpu.VMEM((64,), jnp.int32),
        pltpu.VMEM((64,), jnp.int32),
        pltpu.VMEM((_SEQ, _DIM), jnp.float32),
        pltpu.VMEM((_SEQ, _DIM), jnp.float32),
        pltpu.VMEM((1, _DIM), jnp.float32),
        pltpu.SemaphoreType.DMA,
        pltpu.SemaphoreType.DMA,
        pltpu.SemaphoreType.DMA,
        pltpu.SemaphoreType.DMA,
    ],
    compiler_params=pltpu.CompilerParams(use_tc_tiling_on_sc=False,
                                         needs_layout_passes=False),
)
def _prompt_kernel(tok_hbm, table_hbm, ctx_hbm, out_hbm,
                   tok_v, idx0_v, idx1_v, buf0_v, buf1_v, ctx15_v,
                   sg0, sg1, sw0, sw1):
    wid = lax.axis_index("s") * _NC + lax.axis_index("c")
    base_c = wid * _CPW
    pltpu.sync_copy(tok_hbm.at[pl.ds(base_c, _CPW)], tok_v)
    pltpu.sync_copy(ctx_hbm, buf0_v.at[pl.ds(1, _N_CTX)])
    pltpu.sync_copy(ctx_hbm, buf1_v.at[pl.ds(1, _N_CTX)])
    pltpu.sync_copy(ctx_hbm.at[pl.ds(_N_CTX - 1, 1)], ctx15_v)

    idxs = (idx0_v, idx1_v)
    bufs = (buf0_v, buf1_v)
    sgs = (sg0, sg1)
    sws = (sw0, sw1)

    i16 = lax.iota(jnp.int32, 16)
    pos = [
        jnp.where((i16 + 16 * j) == 0, 0,
                  jnp.minimum(i16 + 16 * j + _N_CTX, _SEQ - 1))
        for j in range(4)
    ]

    def gstart(ci, b):
        cvec = jnp.full((16,), ci, jnp.int32)
        for j in range(4):
            idxs[b][pl.ds(16 * j, 16)] = plsc.load_gather(
                tok_v, [cvec, pos[j]])
        pltpu.async_copy(table_hbm.at[idxs[b].at[pl.ds(0, _NIDX)]],
                         bufs[b].at[pl.ds(_N_CTX, _NIDX)], sgs[b])

    def gwait(b):
        pltpu.make_async_copy(table_hbm.at[idxs[b].at[pl.ds(0, _NIDX)]],
                              bufs[b].at[pl.ds(_N_CTX, _NIDX)],
                              sgs[b]).wait()

    def wstart(ci, b):
        for k in range(_DIM // 16):
            sl = pl.ds(16 * k, 16)
            bufs[b][0, sl] = bufs[b][_N_CTX, sl]
            bufs[b][_N_CTX, sl] = ctx15_v[0, sl]
        pltpu.async_copy(bufs[b], out_hbm.at[base_c + ci], sws[b])

    def wwait(b):
        pltpu.make_async_copy(bufs[b], out_hbm.at[base_c], sws[b]).wait()

    gstart(0, 0)
    gstart(1, 1)
    gwait(0)
    wstart(0, 0)

    def body(g, carry):
        ci = 2 + 2 * g
        wwait(0)
        gstart(ci, 0)
        gwait(1)
        wstart(ci - 1, 1)
        wwait(1)
        gstart(ci + 1, 1)
        gwait(0)
        wstart(ci, 0)
        return carry

    lax.fori_loop(0, (_CPW - 2) // 2, body, 0)

    gwait(1)
    wstart(_CPW - 1, 1)
    wwait(0)
    wwait(1)


def kernel(tokenized_prompts, token_embedding, ctx):
    return _prompt_kernel(tokenized_prompts, token_embedding,
                          ctx.astype(jnp.float32))


# speedup vs baseline: 2.5411x; 1.7266x over previous
import functools

import jax
import jax.numpy as jnp
from jax import lax
from jax.experimental import pallas as pl
from jax.experimental.pallas import tpu as pltpu
from jax.experimental.pallas import tpu_sc as plsc

_N_CLS = 1024
_N_CTX = 16
_DIM = 768
_SEQ = 77
_NC = 2
_NS = 16
_NW = _NC * _NS
_CPW = _N_CLS // _NW
_MAIN = 56
_TAIL = _SEQ - 72


_mesh = plsc.VectorSubcoreMesh(core_axis_name="c", subcore_axis_name="s")


@functools.partial(
    pl.kernel,
    mesh=_mesh,
    out_type=jax.ShapeDtypeStruct((_N_CLS, _SEQ, _DIM), jnp.float32),
    scratch_types=[
        pltpu.VMEM((_CPW, _SEQ), jnp.int32),
        pltpu.VMEM((64,), jnp.int32),
        pltpu.VMEM((_TAIL,), jnp.int32),
        pltpu.VMEM((32,), jnp.int32),
        pltpu.VMEM((72, _DIM), jnp.float32),
        pltpu.VMEM((_TAIL, _DIM), jnp.float32),
        pltpu.VMEM((8, _DIM), jnp.float32),
        pltpu.SemaphoreType.DMA,
    ],
    compiler_params=pltpu.CompilerParams(needs_layout_passes=False),
)
def _prompt_kernel(tok_hbm, table_hbm, ctx_hbm, out_hbm,
                   tok_v, idx_v, tidx_v, cidx_v, buf_v, tail_v, ctx15_v, sem):
    wid = lax.axis_index("s") * _NC + lax.axis_index("c")
    base_c = wid * _CPW
    pltpu.sync_copy(tok_hbm.at[pl.ds(base_c, _CPW)], tok_v)

    i16 = lax.iota(jnp.int32, 16)
    cidx_v[pl.ds(0, 16)] = jnp.maximum(i16 - 1, 0)
    cidx_v[pl.ds(16, 16)] = jnp.full((16,), _N_CTX - 1, jnp.int32)
    pltpu.async_copy(ctx_hbm.at[cidx_v.at[pl.ds(0, 24)]],
                     buf_v.at[pl.ds(0, 24)], sem).wait()
    pltpu.async_copy(ctx_hbm.at[cidx_v.at[pl.ds(24, 8)]],
                     ctx15_v, sem).wait()

    pos = [
        jnp.where((i16 + 16 * j) == 0, 0,
                  jnp.minimum(i16 + 16 * j + _N_CTX, _SEQ - 1))
        for j in range(4)
    ]
    tpos = jnp.minimum(i16 + 72, _SEQ - 1)

    def body(ci, carry):
        cvec = jnp.full((16,), ci, jnp.int32)
        for j in range(4):
            idx_v[pl.ds(16 * j, 16)] = plsc.load_gather(tok_v, [cvec, pos[j]])
        tvals = plsc.load_gather(tok_v, [cvec, tpos])
        plsc.store_scatter(tidx_v, [i16], tvals, mask=i16 < _TAIL)
        pltpu.async_copy(table_hbm.at[idx_v.at[pl.ds(0, _MAIN)]],
                         buf_v.at[pl.ds(_N_CTX, _MAIN)], sem).wait()
        pltpu.async_copy(table_hbm.at[tidx_v], tail_v, sem).wait()
        for k in range(_DIM // 16):
            sl = pl.ds(16 * k, 16)
            buf_v[0, sl] = buf_v[_N_CTX, sl]
            buf_v[_N_CTX, sl] = ctx15_v[0, sl]
        c = base_c + ci
        pltpu.sync_copy(buf_v, out_hbm.at[c, pl.ds(0, 72)])
        pltpu.sync_copy(tail_v, out_hbm.at[c, pl.ds(72, _TAIL)])
        return carry

    lax.fori_loop(0, _CPW, body, 0)


def kernel(tokenized_prompts, token_embedding, ctx):
    return _prompt_kernel(tokenized_prompts, token_embedding,
                          ctx.astype(jnp.float32))
